# baseline (speedup 1.0000x reference)
import functools

import jax
import jax.numpy as jnp
from jax import lax
from jax.experimental import pallas as pl
from jax.experimental.pallas import tpu as pltpu
from jax.experimental.pallas import tpu_sc as plsc

USER = 5000
ITEM = 5000
N = USER + ITEM
D = 128
E = 320000
ANCHOR = 16
GTW = 0.5

NC = 2
NS = 16
NW = NC * NS
CHUNK = 128
ROWS = 2560
EPAD = ROWS * CHUNK
RPW = ROWS // NW
ACC_ROWS = 10240
GROUPS = N // ANCHOR


def _make_sc_spmm(num_phases):
  mesh = plsc.VectorSubcoreMesh(core_axis_name="c", subcore_axis_name="s")

  @functools.partial(
      pl.kernel,
      out_type=jax.ShapeDtypeStruct((num_phases, NC, ACC_ROWS, D),
                                    jnp.float32),
      mesh=mesh,
      scratch_types=[
          pltpu.VMEM_SHARED((ACC_ROWS, D), jnp.float32),
          pltpu.VMEM((RPW // 2, CHUNK), jnp.int32),
          pltpu.VMEM((RPW // 2, CHUNK), jnp.int32),
          pltpu.VMEM((CHUNK, D), jnp.float32),
          pltpu.VMEM((CHUNK, D), jnp.float32),
          pltpu.SemaphoreType.DMA,
          pltpu.SemaphoreType.DMA,
          pltpu.SemaphoreType.DMA,
          pltpu.SemaphoreType.DMA,
      ],
  )
  def k(table, srcs, dsts, zeros, out, acc, sidx, didx, b0, b1, s0, s1,
        t0, t1):
    c = lax.axis_index("c")
    s = lax.axis_index("s")
    wid = s * NC + c

    rows_per_tile = ACC_ROWS // NS
    my_off = s * rows_per_tile

    pltpu.sync_copy(zeros, acc.at[pl.ds(my_off, rows_per_tile)])
    plsc.subcore_barrier()

    for p in range(num_phases):
      for h in range(2):
        base = p * ROWS + wid * RPW + h * (RPW // 2)
        pltpu.sync_copy(srcs.at[pl.ds(base, RPW // 2)], sidx)
        pltpu.sync_copy(dsts.at[pl.ds(base, RPW // 2)], didx)

        H = CHUNK // 2

        def gath(r, buf, sem):
          pltpu.async_copy(table.at[sidx.at[r, pl.ds(0, H)]],
                           buf.at[pl.ds(0, H)], sem)
          pltpu.async_copy(table.at[sidx.at[r, pl.ds(H, H)]],
                           buf.at[pl.ds(H, H)], sem)

        def gwait(r, buf, sem):
          pltpu.make_async_copy(table.at[sidx.at[r, pl.ds(0, H)]],
                                buf.at[pl.ds(0, H)], sem).wait()
          pltpu.make_async_copy(table.at[sidx.at[r, pl.ds(H, H)]],
                                buf.at[pl.ds(H, H)], sem).wait()

        gath(0, b0, s0)

        def chunk(j, carry):
          a0 = 2 * j
          a1 = a0 + 1
          gwait(a0, b0, s0)
          gath(a1, b1, s1)
          pltpu.sync_copy(b0, acc.at[didx.at[a0]], add=True)
          gwait(a1, b1, s1)

          @pl.when(j < RPW // 4 - 1)
          def _():
            gath(a0 + 2, b0, s0)

          pltpu.sync_copy(b1, acc.at[didx.at[a1]], add=True)
          return carry

        lax.fori_loop(0, RPW // 4, chunk, 0)

      plsc.subcore_barrier()
      pltpu.sync_copy(acc.at[pl.ds(my_off, rows_per_tile)],
                      out.at[p, c, pl.ds(my_off, rows_per_tile)])
      if p < num_phases - 1:
        pltpu.sync_copy(zeros, acc.at[pl.ds(my_off, rows_per_tile)])
        plsc.subcore_barrier()

  return k


_sc_spmm2 = _make_sc_spmm(2)
_sc_spmm1 = _make_sc_spmm(1)


def _plane(p, c, rows=N // 5, gridded=True):
  if gridded:
    return pl.BlockSpec((1, 1, rows, D), lambda i: (p, c, i, 0))
  return pl.BlockSpec((1, 1, rows, D), lambda i: (p, c, 0, 0))


def _combine2(k4, p):
  def body(ar, br, outr):
    outr[...] = ar[0, 0] + br[0, 0]

  return pl.pallas_call(
      body,
      grid=(5,),
      in_specs=[_plane(p, 0), _plane(p, 1)],
      out_specs=pl.BlockSpec((N // 5, D), lambda i: (i, 0)),
      out_shape=jax.ShapeDtypeStruct((N, D), jnp.float32),
  )(k4, k4)


def _pnn(k4, dists_t, w, b2, aid):
  def body(q0r, q1r, dr, wr, br, aidr, e1pr, pnnr, anch):
    e1p = q0r[0, 0] + q1r[0, 0]
    e1pr[...] = e1p

    def gather_row(a, carry):
      idx = aidr[a]
      anch[pl.ds(a, 1), :] = (q0r[0, 0, pl.ds(idx, 1), :]
                              + q1r[0, 0, pl.ds(idx, 1), :])
      return carry

    lax.fori_loop(0, ANCHOR, gather_row, 0)

    p_raw = jnp.dot(dr[...], anch[...], preferred_element_type=jnp.float32)
    r = jnp.sum(e1p.reshape(GROUPS, ANCHOR, D), axis=1)
    w1 = wr[:D, :]
    w2 = wr[D:, :]
    a_term = jnp.dot(p_raw, w1, preferred_element_type=jnp.float32)
    r2 = jnp.dot(r, w2, preferred_element_type=jnp.float32)
    q_term = jnp.broadcast_to(r2[None], (ANCHOR, GROUPS, D)).reshape(N, D)
    pnnr[...] = (a_term + q_term) * (1.0 / ANCHOR) + br[...]

  return pl.pallas_call(
      body,
      grid=(1,),
      in_specs=[
          _plane(0, 0, rows=N, gridded=False),
          _plane(0, 1, rows=N, gridded=False),
          pl.BlockSpec((N, ANCHOR), lambda i: (0, 0)),
          pl.BlockSpec((2 * D, D), lambda i: (0, 0)),
          pl.BlockSpec((1, D), lambda i: (0, 0)),
          pl.BlockSpec(memory_space=pltpu.SMEM),
      ],
      out_specs=[
          pl.BlockSpec((N, D), lambda i: (0, 0)),
          pl.BlockSpec((N, D), lambda i: (0, 0)),
      ],
      out_shape=[
          jax.ShapeDtypeStruct((N, D), jnp.float32),
          jax.ShapeDtypeStruct((N, D), jnp.float32),
      ],
      scratch_shapes=[pltpu.VMEM((ANCHOR, D), jnp.float32)],
  )(k4, k4, dists_t, w, b2, aid)


def _final(emb, e1, e1p, epnn, k1, k2, k3):
  def body(embr, e1r, e1pr, epnnr, d0r, d1r, c0r, c1r, cc0r, cc1r,
           s0r, s1r, ss0r, ss1r, eor, clr, slr):
    emb_v = embr[...]
    eor[...] = (emb_v + e1r[...] + e1pr[...] + epnnr[...]
                + d0r[0, 0] + d1r[0, 0])
    clr[...] = (emb_v + (1.0 + GTW) * (c0r[0, 0] + c1r[0, 0])
                + cc0r[0, 0] + cc1r[0, 0])
    slr[...] = (emb_v + (1.0 + GTW) * (s0r[0, 0] + s1r[0, 0])
                + ss0r[0, 0] + ss1r[0, 0])

  blk = pl.BlockSpec((N // 5, D), lambda i: (i, 0))
  return pl.pallas_call(
      body,
      grid=(5,),
      in_specs=[blk] * 4 + [_plane(0, 0), _plane(0, 1),
                            _plane(0, 0), _plane(0, 1),
                            _plane(0, 0), _plane(0, 1),
                            _plane(1, 0), _plane(1, 1),
                            _plane(1, 0), _plane(1, 1)],
      out_specs=[blk] * 3,
      out_shape=[jax.ShapeDtypeStruct((N, D), jnp.float32)] * 3,
  )(emb, e1, e1p, epnn, k3, k3, k1, k1, k2, k2, k1, k1, k2, k2)


def _prep_edges(edge_index, pad_src, pad_dst):
  src = jnp.concatenate([edge_index[0], pad_src]).reshape(ROWS, CHUNK)
  dst = jnp.concatenate([edge_index[1], pad_dst]).reshape(ROWS, CHUNK)
  return src, dst


def kernel(uEmbeds, iEmbeds, W_hidden, b_hidden, dists_array,
           enc_edge_index, sub_edge_index, cmp_edge_index, dec_edge_index,
           anchorset_id):
  embeds = jnp.concatenate([uEmbeds, iEmbeds], axis=0)

  npad = EPAD - E
  pad_src = (jnp.arange(npad, dtype=jnp.int32) % N)
  pad_dst = N + (jnp.arange(npad, dtype=jnp.int32) % 16)

  cs, cd = _prep_edges(cmp_edge_index, pad_src, pad_dst)
  ss, sd = _prep_edges(sub_edge_index, pad_src, pad_dst)
  es, ed = _prep_edges(enc_edge_index, pad_src, pad_dst)
  ds_, dd = _prep_edges(dec_edge_index, pad_src, pad_dst)

  src2 = jnp.concatenate([cs, ss], axis=0)
  dst2 = jnp.concatenate([cd, sd], axis=0)

  zrows = jnp.zeros((ACC_ROWS // NS, D), jnp.float32)
  k1e = _sc_spmm1(embeds, es, ed, zrows)
  k1cs = _sc_spmm2(embeds, src2, dst2, zrows)
  e1 = _combine2(k1e, 0)

  k2e = _sc_spmm1(e1, es, ed, zrows)
  k2cs = _sc_spmm2(e1, src2, dst2, zrows)
  e1p, epnn = _pnn(k2e, dists_array.T, W_hidden,
                   b_hidden.reshape(1, D), anchorset_id)

  k3 = _sc_spmm1(epnn, ds_, dd, zrows)

  eo, cl, sl = _final(embeds, e1, e1p, epnn, k1cs, k2cs, k3)
  return (eo[:USER], eo[USER:], cl, sl)


# speedup vs baseline: 1.0239x; 1.0239x over previous
import functools

import jax
import jax.numpy as jnp
from jax import lax
from jax.experimental import pallas as pl
from jax.experimental.pallas import tpu as pltpu
from jax.experimental.pallas import tpu_sc as plsc

USER = 5000
ITEM = 5000
N = USER + ITEM
D = 128
E = 320000
ANCHOR = 16
GTW = 0.5

NC = 2
NS = 16
NW = NC * NS
CHUNK = 128
ROWS = 2560
EPAD = ROWS * CHUNK
RPW = ROWS // NW
ACC_ROWS = 10240
GROUPS = N // ANCHOR


def _make_sc_spmm(num_phases, table2_from=None):
  mesh = plsc.VectorSubcoreMesh(core_axis_name="c", subcore_axis_name="s")
  two_tables = table2_from is not None

  @functools.partial(
      pl.kernel,
      out_type=jax.ShapeDtypeStruct((num_phases, NC, ACC_ROWS, D),
                                    jnp.float32),
      mesh=mesh,
      scratch_types=[
          pltpu.VMEM_SHARED((ACC_ROWS, D), jnp.float32),
          pltpu.VMEM((RPW // 2, CHUNK), jnp.int32),
          pltpu.VMEM((RPW // 2, CHUNK), jnp.int32),
          pltpu.VMEM((CHUNK, D), jnp.float32),
          pltpu.VMEM((CHUNK, D), jnp.float32),
          pltpu.SemaphoreType.DMA,
          pltpu.SemaphoreType.DMA,
          pltpu.SemaphoreType.DMA,
          pltpu.SemaphoreType.DMA,
      ],
  )
  def k(*args):
    if two_tables:
      (tab_a, tab_b, srcs, dsts, out, acc, sidx, didx, b0, b1,
       s0, s1, t0, t1) = args
    else:
      (tab_a, srcs, dsts, out, acc, sidx, didx, b0, b1,
       s0, s1, t0, t1) = args
      tab_b = tab_a
    c = lax.axis_index("c")
    s = lax.axis_index("s")
    wid = s * NC + c

    z16 = jnp.zeros((16,), jnp.float32)
    nzc = ACC_ROWS // CHUNK // NS

    for p in range(num_phases):
      plsc.subcore_barrier()

      def zrow(i, carry):
        for j in range(D // 16):
          b0[i, pl.ds(j * 16, 16)] = z16
        return carry

      lax.fori_loop(0, CHUNK, zrow, 0)
      for t in range(nzc):
        pltpu.sync_copy(b0, acc.at[pl.ds((s * nzc + t) * CHUNK, CHUNK)])
      plsc.subcore_barrier()

      table = tab_b if (two_tables and p >= table2_from) else tab_a
      for h in range(2):
        base = p * ROWS + wid * RPW + h * (RPW // 2)
        pltpu.sync_copy(srcs.at[pl.ds(base, RPW // 2)], sidx)
        pltpu.sync_copy(dsts.at[pl.ds(base, RPW // 2)], didx)

        H = CHUNK // 2

        def gath(r, buf, sem):
          pltpu.async_copy(table.at[sidx.at[r, pl.ds(0, H)]],
                           buf.at[pl.ds(0, H)], sem)
          pltpu.async_copy(table.at[sidx.at[r, pl.ds(H, H)]],
                           buf.at[pl.ds(H, H)], sem)

        def gwait(r, buf, sem):
          pltpu.make_async_copy(table.at[sidx.at[r, pl.ds(0, H)]],
                                buf.at[pl.ds(0, H)], sem).wait()
          pltpu.make_async_copy(table.at[sidx.at[r, pl.ds(H, H)]],
                                buf.at[pl.ds(H, H)], sem).wait()

        gath(0, b0, s0)

        def chunk(j, carry):
          a0 = 2 * j
          a1 = a0 + 1
          gwait(a0, b0, s0)
          gath(a1, b1, s1)
          pltpu.sync_copy(b0, acc.at[didx.at[a0]], add=True)
          gwait(a1, b1, s1)

          @pl.when(j < RPW // 4 - 1)
          def _():
            gath(a0 + 2, b0, s0)

          pltpu.sync_copy(b1, acc.at[didx.at[a1]], add=True)
          return carry

        lax.fori_loop(0, RPW // 4, chunk, 0)

      plsc.subcore_barrier()
      rows_per_tile = ACC_ROWS // NS
      off = s * rows_per_tile
      pltpu.sync_copy(acc.at[pl.ds(off, rows_per_tile)],
                      out.at[p, c, pl.ds(off, rows_per_tile)])

  return k


_sc_spmm1 = _make_sc_spmm(1)
_sc_spmm2 = _make_sc_spmm(2)
_sc_spmm21 = _make_sc_spmm(3, table2_from=2)


def _plane(p, c, rows=N // 5, gridded=True):
  if gridded:
    return pl.BlockSpec((1, 1, rows, D), lambda i: (p, c, i, 0))
  return pl.BlockSpec((1, 1, rows, D), lambda i: (p, c, 0, 0))


def _combine2(k4, p):
  def body(ar, br, outr):
    outr[...] = ar[0, 0] + br[0, 0]

  return pl.pallas_call(
      body,
      grid=(5,),
      in_specs=[_plane(p, 0), _plane(p, 1)],
      out_specs=pl.BlockSpec((N // 5, D), lambda i: (i, 0)),
      out_shape=jax.ShapeDtypeStruct((N, D), jnp.float32),
  )(k4, k4)


def _pnn(k4, dists_t, w, b2, aid):
  def body(q0r, q1r, dr, wr, br, aidr, e1pr, pnnr, anch):
    e1p = q0r[0, 0] + q1r[0, 0]
    e1pr[...] = e1p

    def gather_row(a, carry):
      idx = aidr[a]
      anch[pl.ds(a, 1), :] = (q0r[0, 0, pl.ds(idx, 1), :]
                              + q1r[0, 0, pl.ds(idx, 1), :])
      return carry

    lax.fori_loop(0, ANCHOR, gather_row, 0)

    p_raw = jnp.dot(dr[...], anch[...], preferred_element_type=jnp.float32)
    r = jnp.sum(e1p.reshape(GROUPS, ANCHOR, D), axis=1)
    w1 = wr[:D, :]
    w2 = wr[D:, :]
    a_term = jnp.dot(p_raw, w1, preferred_element_type=jnp.float32)
    r2 = jnp.dot(r, w2, preferred_element_type=jnp.float32)
    q_term = jnp.broadcast_to(r2[None], (ANCHOR, GROUPS, D)).reshape(N, D)
    pnnr[...] = (a_term + q_term) * (1.0 / ANCHOR) + br[...]

  return pl.pallas_call(
      body,
      grid=(1,),
      in_specs=[
          _plane(0, 0, rows=N, gridded=False),
          _plane(0, 1, rows=N, gridded=False),
          pl.BlockSpec((N, ANCHOR), lambda i: (0, 0)),
          pl.BlockSpec((2 * D, D), lambda i: (0, 0)),
          pl.BlockSpec((1, D), lambda i: (0, 0)),
          pl.BlockSpec(memory_space=pltpu.SMEM),
      ],
      out_specs=[
          pl.BlockSpec((N, D), lambda i: (0, 0)),
          pl.BlockSpec((N, D), lambda i: (0, 0)),
      ],
      out_shape=[
          jax.ShapeDtypeStruct((N, D), jnp.float32),
          jax.ShapeDtypeStruct((N, D), jnp.float32),
      ],
      scratch_shapes=[pltpu.VMEM((ANCHOR, D), jnp.float32)],
  )(k4, k4, dists_t, w, b2, aid)


def _final(emb, e1, e1p, epnn, k1, k2, k3):
  def body(embr, e1r, e1pr, epnnr, d0r, d1r, c0r, c1r, cc0r, cc1r,
           s0r, s1r, ss0r, ss1r, eor, clr, slr):
    emb_v = embr[...]
    eor[...] = (emb_v + e1r[...] + e1pr[...] + epnnr[...]
                + d0r[0, 0] + d1r[0, 0])
    clr[...] = (emb_v + (1.0 + GTW) * (c0r[0, 0] + c1r[0, 0])
                + cc0r[0, 0] + cc1r[0, 0])
    slr[...] = (emb_v + (1.0 + GTW) * (s0r[0, 0] + s1r[0, 0])
                + ss0r[0, 0] + ss1r[0, 0])

  blk = pl.BlockSpec((N // 5, D), lambda i: (i, 0))
  return pl.pallas_call(
      body,
      grid=(5,),
      in_specs=[blk] * 4 + [_plane(2, 0), _plane(2, 1),
                            _plane(0, 0), _plane(0, 1),
                            _plane(0, 0), _plane(0, 1),
                            _plane(1, 0), _plane(1, 1),
                            _plane(1, 0), _plane(1, 1)],
      out_specs=[blk] * 3,
      out_shape=[jax.ShapeDtypeStruct((N, D), jnp.float32)] * 3,
  )(emb, e1, e1p, epnn, k3, k3, k1, k1, k2, k2, k1, k1, k2, k2)


def _prep_edges(edge_index, pad_src, pad_dst):
  src = jnp.concatenate([edge_index[0], pad_src]).reshape(ROWS, CHUNK)
  dst = jnp.concatenate([edge_index[1], pad_dst]).reshape(ROWS, CHUNK)
  return src, dst


def kernel(uEmbeds, iEmbeds, W_hidden, b_hidden, dists_array,
           enc_edge_index, sub_edge_index, cmp_edge_index, dec_edge_index,
           anchorset_id):
  embeds = jnp.concatenate([uEmbeds, iEmbeds], axis=0)

  npad = EPAD - E
  pad_src = (jnp.arange(npad, dtype=jnp.int32) % N)
  pad_dst = N + (jnp.arange(npad, dtype=jnp.int32) % 16)

  cs, cd = _prep_edges(cmp_edge_index, pad_src, pad_dst)
  ss, sd = _prep_edges(sub_edge_index, pad_src, pad_dst)
  es, ed = _prep_edges(enc_edge_index, pad_src, pad_dst)
  ds_, dd = _prep_edges(dec_edge_index, pad_src, pad_dst)

  src2 = jnp.concatenate([cs, ss], axis=0)
  dst2 = jnp.concatenate([cd, sd], axis=0)

  k1e = _sc_spmm1(embeds, es, ed)
  k1cs = _sc_spmm2(embeds, src2, dst2)
  e1 = _combine2(k1e, 0)

  k2e = _sc_spmm1(e1, es, ed)
  e1p, epnn = _pnn(k2e, dists_array.T, W_hidden,
                   b_hidden.reshape(1, D), anchorset_id)

  src3 = jnp.concatenate([src2, ds_], axis=0)
  dst3 = jnp.concatenate([dst2, dd], axis=0)
  k23 = _sc_spmm21(e1, epnn, src3, dst3)

  eo, cl, sl = _final(embeds, e1, e1p, epnn, k1cs, k23, k23)
  return (eo[:USER], eo[USER:], cl, sl)
